# baseline (speedup 1.0000x reference)
import jax
import jax.numpy as jnp
from jax import lax
from jax.experimental import pallas as pl
from jax.experimental.pallas import tpu as pltpu
from jax.experimental.pallas import tpu_sc as plsc

_N = 10000
_E = 320000
_D = 128
_NC = 2
_NS = 16
_K = 80
_EPT = _E // _NS
_HALF = 5120
_L = 5376
_TRASH = 5300
_RPT = _L // _NS
_DEGW = 16
_NBUF = 3
_SEG = 10000
_NSEG = _EPT // _SEG
_CAP = _SEG + _K

_mesh = plsc.VectorSubcoreMesh(core_axis_name="c", subcore_axis_name="s")
_params = pltpu.CompilerParams(use_tc_tiling_on_sc=False, needs_layout_passes=False)


def _agg_body(emb_hbm, src_hbm, dst_hbm, part_hbm, degp_hbm,
              src_big, dst_big, comp_src, comp_dst, rows, ones_b, zdeg,
              accum_sh, deg_sh, sem_g, sem_s, sem_d):
    cid = lax.axis_index("c")
    sid = lax.axis_index("s")

    zeros16 = jnp.zeros((16,), jnp.float32)
    ones16 = jnp.ones((16,), jnp.float32)

    def init_ob(i, _):
        ones_b[i, :] = ones16
        zdeg[i, :] = zeros16
        return 0
    lax.fori_loop(0, _K, init_ob, 0)

    def init_zb(k, _):
        rows[0, k // 8, pl.ds((k % 8) * 16, 16)] = zeros16
        return 0
    lax.fori_loop(0, _K * 8, init_zb, 0)

    row0 = sid * _RPT
    for t in range(4):
        pltpu.sync_copy(rows.at[0], accum_sh.at[pl.ds(row0 + t * _K, _K), :])
        pltpu.sync_copy(zdeg, deg_sh.at[pl.ds(row0 + t * _K, _K), :])
    pltpu.sync_copy(rows.at[0].at[pl.ds(0, 16), :],
                    accum_sh.at[pl.ds(row0 + 4 * _K, 16), :])
    pltpu.sync_copy(zdeg.at[pl.ds(0, 16), :],
                    deg_sh.at[pl.ds(row0 + 4 * _K, 16), :])

    lo = cid * _HALF
    trash16 = jnp.full((16,), _TRASH, jnp.int32)

    def start_gather(j, p):
        pltpu.async_copy(emb_hbm.at[comp_src.at[pl.ds(j * _K, _K)]],
                         rows.at[p], sem_g.at[p])

    def wait_gather(p):
        pltpu.make_async_copy(emb_hbm.at[pl.ds(0, _K), :], rows.at[p],
                              sem_g.at[p]).wait()

    def wait_scatter(jj):
        p = jj % _NBUF
        pltpu.make_async_copy(rows.at[p],
                              accum_sh.at[comp_dst.at[pl.ds(jj * _K, _K)]],
                              sem_s.at[p]).wait()

    for seg in range(_NSEG):
        e0 = sid * _EPT + seg * _SEG
        pltpu.sync_copy(src_hbm.at[pl.ds(e0, _SEG)], src_big)
        pltpu.sync_copy(dst_hbm.at[pl.ds(e0, _SEG)], dst_big)

        def filt(g, cnt):
            dvec = dst_big[pl.ds(g * 16, 16)]
            svec = src_big[pl.ds(g * 16, 16)]
            local = dvec - lo
            mask = (local >= 0) & (local < _HALF)
            keep = jnp.where(mask, jnp.int32(0), jnp.int32(1))
            _, sd = plsc.sort_key_val(keep, local)
            _, ss = plsc.sort_key_val(keep, svec)
            comp_dst[pl.ds(cnt, 16)] = sd
            comp_src[pl.ds(cnt, 16)] = ss
            npop = 16 - jnp.sum(keep)
            return cnt + npop
        cnt = lax.fori_loop(0, _SEG // 16, filt, jnp.int32(0))

        for t in range(_K // 16):
            comp_dst[pl.ds(cnt + t * 16, 16)] = trash16
            comp_src[pl.ds(cnt + t * 16, 16)] = jnp.zeros((16,), jnp.int32)
        nc = jnp.maximum((cnt + _K - 1) // _K, 1)

        if seg == 0:
            plsc.subcore_barrier()

        start_gather(0, 0)

        def chunk(j, _):
            p = j % _NBUF
            wait_gather(p)
            pltpu.async_copy(rows.at[p],
                             accum_sh.at[comp_dst.at[pl.ds(j * _K, _K)]],
                             sem_s.at[p], add=True)
            pltpu.async_copy(ones_b,
                             deg_sh.at[comp_dst.at[pl.ds(j * _K, _K)]],
                             sem_d, add=True)

            @pl.when(j + 1 < nc)
            def _():
                @pl.when(j >= _NBUF - 1)
                def _():
                    wait_scatter(j + 1 - _NBUF)
                start_gather(j + 1, (j + 1) % _NBUF)
            return 0
        lax.fori_loop(0, nc, chunk, 0)

        m = jnp.minimum(nc, _NBUF)

        def drain_s(i, _):
            wait_scatter(nc - m + i)
            return 0
        lax.fori_loop(0, m, drain_s, 0)

        def drain_d(j, _):
            pltpu.make_async_copy(degp_hbm.at[pl.ds(0, _K), :], ones_b,
                                  sem_d).wait()
            return 0
        lax.fori_loop(0, nc, drain_d, 0)

    plsc.subcore_barrier()

    out0 = cid * _L + row0
    for t in range(4):
        pltpu.sync_copy(accum_sh.at[pl.ds(row0 + t * _K, _K), :],
                        part_hbm.at[pl.ds(out0 + t * _K, _K), :])
        pltpu.sync_copy(deg_sh.at[pl.ds(row0 + t * _K, _K), :],
                        degp_hbm.at[pl.ds(out0 + t * _K, _K), :])
    pltpu.sync_copy(accum_sh.at[pl.ds(row0 + 4 * _K, 16), :],
                    part_hbm.at[pl.ds(out0 + 4 * _K, 16), :])
    pltpu.sync_copy(deg_sh.at[pl.ds(row0 + 4 * _K, 16), :],
                    degp_hbm.at[pl.ds(out0 + 4 * _K, 16), :])


_agg_kernel = pl.kernel(
    _agg_body,
    out_type=(
        jax.ShapeDtypeStruct((_NC * _L, _D), jnp.float32),
        jax.ShapeDtypeStruct((_NC * _L, _DEGW), jnp.float32),
    ),
    mesh=_mesh,
    scratch_types=[
        pltpu.VMEM((_SEG,), jnp.int32),
        pltpu.VMEM((_SEG,), jnp.int32),
        pltpu.VMEM((_CAP,), jnp.int32),
        pltpu.VMEM((_CAP,), jnp.int32),
        pltpu.VMEM((_NBUF, _K, _D), jnp.float32),
        pltpu.VMEM((_K, _DEGW), jnp.float32),
        pltpu.VMEM((_K, _DEGW), jnp.float32),
        pltpu.VMEM_SHARED((_L, _D), jnp.float32),
        pltpu.VMEM_SHARED((_L, _DEGW), jnp.float32),
        pltpu.SemaphoreType.DMA((_NBUF,)),
        pltpu.SemaphoreType.DMA((_NBUF,)),
        pltpu.SemaphoreType.DMA,
    ],
    compiler_params=_params,
)


def _combine_body(p_ref, d_ref, o_ref):
    s = p_ref[...]
    dg = d_ref[:, 0:1]
    o_ref[...] = jnp.where(dg > 0, s / jnp.maximum(dg, 1.0), 0.0)


_BLK = 128
_NOUT = _NC * _HALF


def _combine(part, degp):
    def pmap(i):
        return (jnp.where(i < _HALF // _BLK, i, i + (_L - _HALF) // _BLK), 0)
    return pl.pallas_call(
        _combine_body,
        grid=(_NOUT // _BLK,),
        in_specs=[
            pl.BlockSpec((_BLK, _D), pmap),
            pl.BlockSpec((_BLK, _DEGW), pmap),
        ],
        out_specs=pl.BlockSpec((_BLK, _D), lambda i: (i, 0)),
        out_shape=jax.ShapeDtypeStruct((_NOUT, _D), jnp.float32),
    )(part, degp)


@jax.jit
def kernel(embeddings, edge_index):
    src = edge_index[0].astype(jnp.int32)
    dst = edge_index[1].astype(jnp.int32)
    part, degp = _agg_kernel(embeddings, src, dst)
    return _combine(part, degp)[:_N]
